# Initial kernel scaffold; baseline (speedup 1.0000x reference)
import jax
import jax.numpy as jnp
from jax.experimental import pallas as pl


def kernel(x, xt, time_emb, return_features, z, params):
    raise NotImplementedError("write your pallas kernel here")


# speedup vs baseline: 8.1764x; 8.1764x over previous
import functools

import jax
import jax.numpy as jnp
from jax.experimental import pallas as pl

B = 8
N0 = 2048
CTX = 768
K = 32


def _lrelu(x):
    return jnp.where(x >= 0, x, 0.01 * x)


def _mm(a, b):
    return jax.lax.dot_general(a, b, (((1,), (0,)), ((), ())),
                               preferred_element_type=jnp.float32)


def _adagn_call(x, ctx3, W, b, Wc, bc, g):
    Bb, N, Cin = x.shape
    Cout = W.shape[0]
    Cg = Cout // g
    wT = W.T
    wcT = Wc.T
    b2 = b[None, :]
    bc2 = bc[None, :]
    G = (jnp.arange(Cout)[:, None] // Cg == jnp.arange(g)[None, :]).astype(jnp.float32)
    GT = G.T

    def body(x_ref, wT_ref, b_ref, ctx_ref, wcT_ref, bc_ref, G_ref, GT_ref, o_ref):
        xv = x_ref[0]
        y = _mm(xv, wT_ref[...]) + b_ref[...]
        Gm = G_ref[...]
        cnt = jnp.float32(N * Cg)
        mu_g = _mm(jnp.sum(y, axis=0, keepdims=True), Gm) / cnt
        mu_c = _mm(mu_g, GT_ref[...])
        sq = (y - mu_c) ** 2
        var_g = _mm(jnp.sum(sq, axis=0, keepdims=True), Gm) / cnt
        var_c = _mm(var_g, GT_ref[...])
        yn = (y - mu_c) / jnp.sqrt(var_c + 1e-5)
        gb = _mm(ctx_ref[0], wcT_ref[...]) + bc_ref[...]
        ga = gb[:, :Cout]
        be = gb[:, Cout:]
        o_ref[0] = _lrelu(yn * (1.0 + ga) + be)

    return pl.pallas_call(
        body,
        grid=(Bb,),
        in_specs=[
            pl.BlockSpec((1, N, Cin), lambda i: (i, 0, 0)),
            pl.BlockSpec((Cin, Cout), lambda i: (0, 0)),
            pl.BlockSpec((1, Cout), lambda i: (0, 0)),
            pl.BlockSpec((1, 1, CTX), lambda i: (i, 0, 0)),
            pl.BlockSpec((CTX, 2 * Cout), lambda i: (0, 0)),
            pl.BlockSpec((1, 2 * Cout), lambda i: (0, 0)),
            pl.BlockSpec((Cout, g), lambda i: (0, 0)),
            pl.BlockSpec((g, Cout), lambda i: (0, 0)),
        ],
        out_specs=pl.BlockSpec((1, N, Cout), lambda i: (i, 0, 0)),
        out_shape=jax.ShapeDtypeStruct((Bb, N, Cout), jnp.float32),
    )(x, wT, b2, ctx3, wcT, bc2, G, GT)


def _down_call(xyzT, nxyz_pad, table, layers):
    Bb, M, Ct = nxyz_pad.shape
    N = xyzT.shape[2]
    w1T, b1 = layers[0]['W'].T, layers[0]['b'][None, :]
    w2T, b2 = layers[1]['W'].T, layers[1]['b'][None, :]
    w3T, b3 = layers[2]['W'].T, layers[2]['b'][None, :]
    C3 = w3T.shape[1]

    def body(xyzT_ref, np_ref, tab_ref, w1T_ref, b1_ref, w2T_ref, b2_ref,
             w3T_ref, b3_ref, o_ref):
        xyzt = xyzT_ref[0]
        npad = np_ref[0]
        d = ((npad[:, 0:1] - xyzt[0:1, :]) ** 2
             + (npad[:, 1:2] - xyzt[1:2, :]) ** 2
             + (npad[:, 2:3] - xyzt[2:3, :]) ** 2)
        iota = jax.lax.broadcasted_iota(jnp.int32, (M, N), 1)
        tab = tab_ref[0]

        def step(_, carry):
            d, acc = carry
            rowmin = jnp.min(d, axis=1, keepdims=True)
            cand = jnp.where(d == rowmin, iota, N)
            jsel = jnp.min(cand, axis=1, keepdims=True)
            onehot = iota == jsel
            oh = onehot.astype(jnp.float32)
            gath = _mm(oh, tab)
            inp = gath - npad
            h = _lrelu(_mm(inp, w1T_ref[...]) + b1_ref[...])
            h = _lrelu(_mm(h, w2T_ref[...]) + b2_ref[...])
            h = _lrelu(_mm(h, w3T_ref[...]) + b3_ref[...])
            acc = jnp.maximum(acc, h)
            d = jnp.where(onehot, jnp.float32(jnp.inf), d)
            return d, acc

        acc0 = jnp.full((M, C3), -jnp.inf, jnp.float32)
        _, acc = jax.lax.fori_loop(0, K, step, (d, acc0))
        o_ref[0] = acc

    return pl.pallas_call(
        body,
        grid=(Bb,),
        in_specs=[
            pl.BlockSpec((1, 3, N), lambda i: (i, 0, 0)),
            pl.BlockSpec((1, M, Ct), lambda i: (i, 0, 0)),
            pl.BlockSpec((1, N, Ct), lambda i: (i, 0, 0)),
            pl.BlockSpec(w1T.shape, lambda i: (0, 0)),
            pl.BlockSpec(b1.shape, lambda i: (0, 0)),
            pl.BlockSpec(w2T.shape, lambda i: (0, 0)),
            pl.BlockSpec(b2.shape, lambda i: (0, 0)),
            pl.BlockSpec(w3T.shape, lambda i: (0, 0)),
            pl.BlockSpec(b3.shape, lambda i: (0, 0)),
        ],
        out_specs=pl.BlockSpec((1, M, C3), lambda i: (i, 0, 0)),
        out_shape=jax.ShapeDtypeStruct((Bb, M, C3), jnp.float32),
    )(xyzT, nxyz_pad, table, w1T, b1, w2T, b2, w3T, b3)


def _fprop_call(xcT, nf, ff, fc, layers):
    Bb, Nf, Cff = ff.shape
    Nc = xcT.shape[2]
    Cfc = fc.shape[2]
    W1 = layers[0]['W']
    w1aT = W1[:, :Cff].T
    w1bT = W1[:, Cff:].T
    b1 = layers[0]['b'][None, :]
    w2T, b2 = layers[1]['W'].T, layers[1]['b'][None, :]
    w3T, b3 = layers[2]['W'].T, layers[2]['b'][None, :]
    C3 = w3T.shape[1]

    def body(xcT_ref, nf_ref, ff_ref, fc_ref, w1aT_ref, w1bT_ref, b1_ref,
             w2T_ref, b2_ref, w3T_ref, b3_ref, o_ref):
        xct = xcT_ref[0]
        nfv = nf_ref[0]
        d = ((nfv[:, 0:1] - xct[0:1, :]) ** 2
             + (nfv[:, 1:2] - xct[1:2, :]) ** 2
             + (nfv[:, 2:3] - xct[2:3, :]) ** 2)
        iota = jax.lax.broadcasted_iota(jnp.int32, (Nf, Nc), 1)
        fcv = fc_ref[0]
        acc = jnp.zeros((Nf, Cfc), jnp.float32)
        wsum = jnp.zeros((Nf, 1), jnp.float32)
        for _ in range(3):
            rowmin = jnp.min(d, axis=1, keepdims=True)
            cand = jnp.where(d == rowmin, iota, Nc)
            jsel = jnp.min(cand, axis=1, keepdims=True)
            onehot = iota == jsel
            oh = onehot.astype(jnp.float32)
            gk = _mm(oh, fcv)
            wk = 1.0 / (rowmin + 1e-8)
            acc = acc + gk * wk
            wsum = wsum + wk
            d = jnp.where(onehot, jnp.float32(jnp.inf), d)
        interp = acc / wsum
        h = _lrelu(_mm(ff_ref[0], w1aT_ref[...]) + _mm(interp, w1bT_ref[...])
                   + b1_ref[...])
        h = _lrelu(_mm(h, w2T_ref[...]) + b2_ref[...])
        h = _lrelu(_mm(h, w3T_ref[...]) + b3_ref[...])
        o_ref[0] = h

    return pl.pallas_call(
        body,
        grid=(Bb,),
        in_specs=[
            pl.BlockSpec((1, 3, Nc), lambda i: (i, 0, 0)),
            pl.BlockSpec((1, Nf, 3), lambda i: (i, 0, 0)),
            pl.BlockSpec((1, Nf, Cff), lambda i: (i, 0, 0)),
            pl.BlockSpec((1, Nc, Cfc), lambda i: (i, 0, 0)),
            pl.BlockSpec(w1aT.shape, lambda i: (0, 0)),
            pl.BlockSpec(w1bT.shape, lambda i: (0, 0)),
            pl.BlockSpec(b1.shape, lambda i: (0, 0)),
            pl.BlockSpec(w2T.shape, lambda i: (0, 0)),
            pl.BlockSpec(b2.shape, lambda i: (0, 0)),
            pl.BlockSpec(w3T.shape, lambda i: (0, 0)),
            pl.BlockSpec(b3.shape, lambda i: (0, 0)),
        ],
        out_specs=pl.BlockSpec((1, Nf, C3), lambda i: (i, 0, 0)),
        out_shape=jax.ShapeDtypeStruct((Bb, Nf, C3), jnp.float32),
    )(xcT, nf, ff, fc, w1aT, w1bT, b1, w2T, b2, w3T, b3)


def _attn_call(x, p):
    Bb, M, C = x.shape
    wqT, bq = p['Wq'].T, p['bq'][None, :]
    wkT, bk = p['Wk'].T, p['bk'][None, :]
    wvT, bv = p['Wv'].T, p['bv'][None, :]
    woT, bo = p['Wo'].T, p['bo'][None, :]

    def body(x_ref, wqT_ref, bq_ref, wkT_ref, bk_ref, wvT_ref, bv_ref,
             woT_ref, bo_ref, o_ref):
        xv = x_ref[0]
        q = _mm(xv, wqT_ref[...]) + bq_ref[...]
        k = _mm(xv, wkT_ref[...]) + bk_ref[...]
        v = _mm(xv, wvT_ref[...]) + bv_ref[...]
        s = jax.lax.dot_general(q, k, (((1,), (1,)), ((), ())),
                                preferred_element_type=jnp.float32)
        s = s / jnp.sqrt(jnp.float32(512.0))
        smax = jnp.max(s, axis=1, keepdims=True)
        e = jnp.exp(s - smax)
        a = e / jnp.sum(e, axis=1, keepdims=True)
        o = _mm(a, v)
        o_ref[0] = xv + _mm(o, woT_ref[...]) + bo_ref[...]

    return pl.pallas_call(
        body,
        grid=(Bb,),
        in_specs=[
            pl.BlockSpec((1, M, C), lambda i: (i, 0, 0)),
            pl.BlockSpec(wqT.shape, lambda i: (0, 0)),
            pl.BlockSpec(bq.shape, lambda i: (0, 0)),
            pl.BlockSpec(wkT.shape, lambda i: (0, 0)),
            pl.BlockSpec(bk.shape, lambda i: (0, 0)),
            pl.BlockSpec(wvT.shape, lambda i: (0, 0)),
            pl.BlockSpec(bv.shape, lambda i: (0, 0)),
            pl.BlockSpec(woT.shape, lambda i: (0, 0)),
            pl.BlockSpec(bo.shape, lambda i: (0, 0)),
        ],
        out_specs=pl.BlockSpec((1, M, C), lambda i: (i, 0, 0)),
        out_shape=jax.ShapeDtypeStruct((Bb, M, C), jnp.float32),
    )(x, wqT, bq, wkT, bk, wvT, bv, woT, bo)


def _pred_call(f0flat, p):
    R = f0flat.shape[0]
    T = 8
    Rt = R // T
    w1T = p['W1'].T
    b1 = p['b1'][None, :]
    w2T = p['W2'].T
    b2 = p['b2'][None, :]
    g2 = p['g'][None, :]
    be2 = p['be'][None, :]

    def body_a(x_ref, w1T_ref, b1_ref, h_ref, s_ref, q_ref):
        h = _mm(x_ref[...], w1T_ref[...]) + b1_ref[...]
        h_ref[...] = h
        s_ref[0] = jnp.sum(h, axis=0, keepdims=True)
        q_ref[0] = jnp.sum(h * h, axis=0, keepdims=True)

    h, s, q = pl.pallas_call(
        body_a,
        grid=(T,),
        in_specs=[
            pl.BlockSpec((Rt, 256), lambda i: (i, 0)),
            pl.BlockSpec((256, 512), lambda i: (0, 0)),
            pl.BlockSpec((1, 512), lambda i: (0, 0)),
        ],
        out_specs=[
            pl.BlockSpec((Rt, 512), lambda i: (i, 0)),
            pl.BlockSpec((1, 1, 512), lambda i: (i, 0, 0)),
            pl.BlockSpec((1, 1, 512), lambda i: (i, 0, 0)),
        ],
        out_shape=[
            jax.ShapeDtypeStruct((R, 512), jnp.float32),
            jax.ShapeDtypeStruct((T, 1, 512), jnp.float32),
            jax.ShapeDtypeStruct((T, 1, 512), jnp.float32),
        ],
    )(f0flat, w1T, b1)

    def body_b(h_ref, s_ref, q_ref, g_ref, be_ref, w2T_ref, b2_ref, o_ref):
        cnt = jnp.float32(R)
        m = jnp.sum(s_ref[:, 0, :], axis=0, keepdims=True) / cnt
        var = jnp.sum(q_ref[:, 0, :], axis=0, keepdims=True) / cnt - m * m
        hv = h_ref[...]
        hn = g_ref[...] * (hv - m) / jnp.sqrt(var + 1e-5) + be_ref[...]
        hl = _lrelu(hn)
        o_ref[...] = _mm(hl, w2T_ref[...]) + b2_ref[...]

    out = pl.pallas_call(
        body_b,
        grid=(T,),
        in_specs=[
            pl.BlockSpec((Rt, 512), lambda i: (i, 0)),
            pl.BlockSpec((T, 1, 512), lambda i: (0, 0, 0)),
            pl.BlockSpec((T, 1, 512), lambda i: (0, 0, 0)),
            pl.BlockSpec((1, 512), lambda i: (0, 0)),
            pl.BlockSpec((1, 512), lambda i: (0, 0)),
            pl.BlockSpec((512, 3), lambda i: (0, 0)),
            pl.BlockSpec((1, 3), lambda i: (0, 0)),
        ],
        out_specs=pl.BlockSpec((Rt, 3), lambda i: (i, 0)),
        out_shape=jax.ShapeDtypeStruct((R, 3), jnp.float32),
    )(h, s, q, g2, be2, w2T, b2)
    return out


def _pad_pts(pts, Ct):
    Bb, M, _ = pts.shape
    return jnp.concatenate([pts, jnp.zeros((Bb, M, Ct - 3), jnp.float32)], axis=2)


@jax.jit
def kernel(x, xt, time_emb, return_features, z, params):
    del x, return_features
    ctx3 = jnp.concatenate([z, time_emb], axis=1)[:, None, :]

    xtT = xt
    x1T = xtT[:, :, ::2]
    x2T = x1T[:, :, ::4]
    x3T = x2T[:, :, ::8]
    ptsT = lambda a: a.transpose(0, 2, 1)
    xt_p, x1_p, x2_p, x3_p = map(ptsT, (xtT, x1T, x2T, x3T))

    f0 = _adagn_call(xt_p, ctx3, params['an0']['W'], params['an0']['b'],
                     params['an0']['Wc'], params['an0']['bc'], 8)

    tab1 = jnp.concatenate([xt_p, f0], axis=2)
    f1 = _down_call(xtT, _pad_pts(x1_p, 67), tab1, params['down1'])
    f1 = _adagn_call(f1, ctx3, params['an1']['W'], params['an1']['b'],
                     params['an1']['Wc'], params['an1']['bc'], 8)

    tab2 = jnp.concatenate([x1_p, f1], axis=2)
    f2 = _down_call(x1T, _pad_pts(x2_p, 131), tab2, params['down2'])
    f2 = _adagn_call(f2, ctx3, params['an2']['W'], params['an2']['b'],
                     params['an2']['Wc'], params['an2']['bc'], 16)

    tab3 = jnp.concatenate([x2_p, f2], axis=2)
    f3 = _down_call(x2T, _pad_pts(x3_p, 259), tab3, params['down3'])
    f3 = _adagn_call(f3, ctx3, params['an3']['W'], params['an3']['b'],
                     params['an3']['Wc'], params['an3']['bc'], 32)

    f3 = _attn_call(f3, params['attn'])

    f2 = _fprop_call(x3T, x2_p, f2, f3, params['up1'])
    f2 = _adagn_call(f2, ctx3, params['an4']['W'], params['an4']['b'],
                     params['an4']['Wc'], params['an4']['bc'], 16)

    f1 = _fprop_call(x2T, x1_p, f1, f2, params['up2'])
    f1 = _adagn_call(f1, ctx3, params['an5']['W'], params['an5']['b'],
                     params['an5']['Wc'], params['an5']['bc'], 8)

    f0 = _fprop_call(x1T, xt_p, f0, f1, params['up3'])
    f0 = _adagn_call(f0, ctx3, params['an6']['W'], params['an6']['b'],
                     params['an6']['Wc'], params['an6']['bc'], 16)

    out = _pred_call(f0.reshape(B * N0, 256), params['pred'])
    return out.reshape(B, N0, 3).transpose(0, 2, 1)
